# Initial kernel scaffold; baseline (speedup 1.0000x reference)
import jax
import jax.numpy as jnp
from jax.experimental import pallas as pl


def kernel(xyz, features, W1, b1, g1, beta1, W2, b2, g2, beta2, W3, b3, g3, beta3):
    raise NotImplementedError("write your pallas kernel here")


# speedup vs baseline: 1.1743x; 1.1743x over previous
import functools

import jax
import jax.numpy as jnp
from jax.experimental import pallas as pl
from jax.experimental.pallas import tpu as pltpu

NPOINT = 1024
RADIUS = 0.4
NSAMPLE = 64


def _fps_body(xs_ref, ys_ref, zs_ref, out_ref):
    B, N = xs_ref.shape
    xs = xs_ref[:, :]
    ys = ys_ref[:, :]
    zs = zs_ref[:, :]
    iota = jax.lax.broadcasted_iota(jnp.int32, (B, N), 1)

    def step(k, carry):
        dists, f = carry
        out_ref[pl.ds(k, 1), :] = jnp.transpose(f)
        sel = iota == f
        cx = jnp.sum(jnp.where(sel, xs, 0.0), axis=1, keepdims=True)
        cy = jnp.sum(jnp.where(sel, ys, 0.0), axis=1, keepdims=True)
        cz = jnp.sum(jnp.where(sel, zs, 0.0), axis=1, keepdims=True)
        dx = xs - cx
        dy = ys - cy
        dz = zs - cz
        d = dx * dx + dy * dy + dz * dz
        dists = jnp.minimum(dists, d)
        m = jnp.max(dists, axis=1, keepdims=True)
        fn = jnp.min(jnp.where(dists == m, iota, N), axis=1, keepdims=True)
        return dists, fn.astype(jnp.int32)

    dists0 = jnp.full((B, N), 1e10, dtype=jnp.float32)
    f0 = jnp.zeros((B, 1), dtype=jnp.int32)
    jax.lax.fori_loop(0, NPOINT, step, (dists0, f0))


def _fps_pallas(xyz):
    B, N, _ = xyz.shape
    xs = xyz[:, :, 0]
    ys = xyz[:, :, 1]
    zs = xyz[:, :, 2]
    out = pl.pallas_call(
        _fps_body,
        out_shape=jax.ShapeDtypeStruct((NPOINT, B), jnp.int32),
    )(xs, ys, zs)
    return jnp.transpose(out)


def _ball_query_idx(new_xyz, xyz, radius, K):
    d2 = jnp.sum((new_xyz[:, :, None, :] - xyz[:, None, :, :]) ** 2, axis=-1)
    N = xyz.shape[1]
    scores = jnp.where(d2 < radius * radius,
                       jnp.arange(N, dtype=jnp.int32)[None, None, :], jnp.int32(N))
    neg_vals, _ = jax.lax.top_k(-scores, K)
    idx = -neg_vals
    idx = jnp.where(idx >= N, 0, idx)
    return idx


def _conv_bn_relu(x, W, b, g, beta):
    x = jnp.einsum('oc,bcsp->bosp', W, x) + b[None, :, None, None]
    m = jnp.mean(x, axis=(0, 2, 3), keepdims=True)
    v = jnp.var(x, axis=(0, 2, 3), keepdims=True)
    x = (x - m) / jnp.sqrt(v + 1e-5) * g[None, :, None, None] + beta[None, :, None, None]
    return jax.nn.relu(x)


def kernel(xyz, features, W1, b1, g1, beta1, W2, b2, g2, beta2, W3, b3, g3, beta3):
    fidx = _fps_pallas(xyz)
    new_xyz = jax.vmap(lambda p, i: p[i])(xyz, fidx)
    idx = _ball_query_idx(new_xyz, xyz, RADIUS, NSAMPLE)
    grouped_xyz = jax.vmap(lambda p, i: p[i])(xyz, idx)
    grouped_xyz = grouped_xyz - new_xyz[:, :, None, :]
    gx = jnp.transpose(grouped_xyz, (0, 3, 2, 1))
    gf = jax.vmap(lambda f, i: f[:, i])(features, idx)
    gf = jnp.transpose(gf, (0, 1, 3, 2))
    grouped = jnp.concatenate([gx, gf], axis=1)
    x = _conv_bn_relu(grouped, W1, b1, g1, beta1)
    x = _conv_bn_relu(x, W2, b2, g2, beta2)
    x = _conv_bn_relu(x, W3, b3, g3, beta3)
    new_features = jnp.max(x, axis=2)
    return new_xyz, new_features


# speedup vs baseline: 2.0056x; 1.7080x over previous
import functools

import jax
import jax.numpy as jnp
from jax.experimental import pallas as pl
from jax.experimental.pallas import tpu as pltpu

B = 8
N = 4096
NPOINT = 1024
RADIUS = 0.4
NSAMPLE = 64
R2 = jnp.float32(RADIUS * RADIUS)
M = B * NPOINT * NSAMPLE
TILE = 512
GRID = M // TILE
EPS = 1e-5


def _fps_body(xs_ref, ys_ref, zs_ref, out_ref):
    nb, n = xs_ref.shape
    xs = xs_ref[:, :]
    ys = ys_ref[:, :]
    zs = zs_ref[:, :]
    iota = jax.lax.broadcasted_iota(jnp.int32, (nb, n), 1)

    def step(k, carry):
        dists, f = carry
        out_ref[pl.ds(k, 1), :] = jnp.transpose(f)
        sel = iota == f
        cx = jnp.sum(jnp.where(sel, xs, 0.0), axis=1, keepdims=True)
        cy = jnp.sum(jnp.where(sel, ys, 0.0), axis=1, keepdims=True)
        cz = jnp.sum(jnp.where(sel, zs, 0.0), axis=1, keepdims=True)
        dx = xs - cx
        dy = ys - cy
        dz = zs - cz
        d = dx * dx + dy * dy + dz * dz
        dists = jnp.minimum(dists, d)
        m = jnp.max(dists, axis=1, keepdims=True)
        fn = jnp.min(jnp.where(dists == m, iota, n), axis=1, keepdims=True)
        return dists, fn.astype(jnp.int32)

    dists0 = jnp.full((nb, n), 1e10, dtype=jnp.float32)
    f0 = jnp.zeros((nb, 1), dtype=jnp.int32)
    jax.lax.fori_loop(0, NPOINT, step, (dists0, f0))


def _fps_pallas(xyz):
    out = pl.pallas_call(
        _fps_body,
        out_shape=jax.ShapeDtypeStruct((NPOINT, B), jnp.int32),
    )(xyz[:, :, 0], xyz[:, :, 1], xyz[:, :, 2])
    return jnp.transpose(out)


def _ptconv_body(x_ref, w_ref, b_ref, o_ref):
    o_ref[0] = jnp.dot(x_ref[0], w_ref[:, :],
                       preferred_element_type=jnp.float32) + b_ref[:, :]


def _ptconv(p_in, w1pad, b1):
    return pl.pallas_call(
        _ptconv_body,
        grid=(B, N // TILE),
        in_specs=[
            pl.BlockSpec((1, TILE, 136), lambda b, i: (b, i, 0)),
            pl.BlockSpec((136, 128), lambda b, i: (0, 0)),
            pl.BlockSpec((1, 128), lambda b, i: (0, 0)),
        ],
        out_specs=pl.BlockSpec((1, TILE, 128), lambda b, i: (b, i, 0)),
        out_shape=jax.ShapeDtypeStruct((B, N, 128), jnp.float32),
    )(p_in, w1pad, b1.reshape(1, 128))


def _u_body(x_ref, w_ref, o_ref):
    o_ref[:, :] = jnp.dot(x_ref[:, :], w_ref[:, :],
                          preferred_element_type=jnp.float32)


def _u_mat(nx_pad, w1xpad):
    return pl.pallas_call(
        _u_body,
        grid=(B * NPOINT // TILE,),
        in_specs=[
            pl.BlockSpec((TILE, 8), lambda i: (i, 0)),
            pl.BlockSpec((8, 128), lambda i: (0, 0)),
        ],
        out_specs=pl.BlockSpec((TILE, 128), lambda i: (i, 0)),
        out_shape=jax.ShapeDtypeStruct((B * NPOINT, 128), jnp.float32),
    )(nx_pad, w1xpad)


def _stats1_body(g_ref, u_ref, sum_ref, sq_ref):
    i = pl.program_id(0)
    y1 = (g_ref[:, :].reshape(8, NSAMPLE, 128)
          - u_ref[:, :].reshape(8, 1, 128)).reshape(TILE, 128)
    ps = jnp.sum(y1.reshape(64, 8, 128), axis=0)
    pq = jnp.sum((y1 * y1).reshape(64, 8, 128), axis=0)

    @pl.when(i == 0)
    def _():
        sum_ref[:, :] = jnp.zeros_like(sum_ref)
        sq_ref[:, :] = jnp.zeros_like(sq_ref)

    sum_ref[:, :] += ps
    sq_ref[:, :] += pq


def _stats1(g, u):
    return pl.pallas_call(
        _stats1_body,
        grid=(GRID,),
        in_specs=[
            pl.BlockSpec((TILE, 128), lambda i: (i, 0)),
            pl.BlockSpec((8, 128), lambda i: (i, 0)),
        ],
        out_specs=[
            pl.BlockSpec((8, 128), lambda i: (0, 0)),
            pl.BlockSpec((8, 128), lambda i: (0, 0)),
        ],
        out_shape=[jax.ShapeDtypeStruct((8, 128), jnp.float32)] * 2,
    )(g, u)


def _affine(sum_, sq_, gamma, beta):
    mean = jnp.sum(sum_, axis=0, keepdims=True) / M
    var = jnp.sum(sq_, axis=0, keepdims=True) / M - mean * mean
    scale = gamma.reshape(1, -1) / jnp.sqrt(var + EPS)
    shift = beta.reshape(1, -1) - mean * scale
    return scale, shift


def _mlp2_body(g_ref, u_ref, sum_ref, sq_ref, ga_ref, be_ref, w2_ref,
               x1_ref, sum2_ref, sq2_ref):
    i = pl.program_id(0)
    scale, shift = _affine(sum_ref[:, :], sq_ref[:, :], ga_ref[:, :], be_ref[:, :])
    y1 = (g_ref[:, :].reshape(8, NSAMPLE, 128)
          - u_ref[:, :].reshape(8, 1, 128)).reshape(TILE, 128)
    x1 = jnp.maximum(y1 * scale + shift, 0.0)
    x1_ref[:, :] = x1
    y2 = jnp.dot(x1, w2_ref[:, :], preferred_element_type=jnp.float32)
    ps = jnp.sum(y2.reshape(64, 8, 256), axis=0)
    pq = jnp.sum((y2 * y2).reshape(64, 8, 256), axis=0)

    @pl.when(i == 0)
    def _():
        sum2_ref[:, :] = jnp.zeros_like(sum2_ref)
        sq2_ref[:, :] = jnp.zeros_like(sq2_ref)

    sum2_ref[:, :] += ps
    sq2_ref[:, :] += pq


def _mlp2(g, u, s1, q1, g1, beta1, w2t):
    return pl.pallas_call(
        _mlp2_body,
        grid=(GRID,),
        in_specs=[
            pl.BlockSpec((TILE, 128), lambda i: (i, 0)),
            pl.BlockSpec((8, 128), lambda i: (i, 0)),
            pl.BlockSpec((8, 128), lambda i: (0, 0)),
            pl.BlockSpec((8, 128), lambda i: (0, 0)),
            pl.BlockSpec((1, 128), lambda i: (0, 0)),
            pl.BlockSpec((1, 128), lambda i: (0, 0)),
            pl.BlockSpec((128, 256), lambda i: (0, 0)),
        ],
        out_specs=[
            pl.BlockSpec((TILE, 128), lambda i: (i, 0)),
            pl.BlockSpec((8, 256), lambda i: (0, 0)),
            pl.BlockSpec((8, 256), lambda i: (0, 0)),
        ],
        out_shape=[
            jax.ShapeDtypeStruct((M, 128), jnp.float32),
            jax.ShapeDtypeStruct((8, 256), jnp.float32),
            jax.ShapeDtypeStruct((8, 256), jnp.float32),
        ],
    )(g, u, s1, q1, g1.reshape(1, 128), beta1.reshape(1, 128), w2t)


def _mlp3_body(x1_ref, sum2_ref, sq2_ref, ga_ref, be_ref, w2_ref, w3_ref,
               mx_ref, mn_ref, sum3_ref, sq3_ref):
    i = pl.program_id(0)
    scale, shift = _affine(sum2_ref[:, :], sq2_ref[:, :], ga_ref[:, :], be_ref[:, :])
    y2 = jnp.dot(x1_ref[:, :], w2_ref[:, :], preferred_element_type=jnp.float32)
    x2 = jnp.maximum(y2 * scale + shift, 0.0)
    y3 = jnp.dot(x2, w3_ref[:, :], preferred_element_type=jnp.float32)
    y3g = y3.reshape(8, NSAMPLE, 256)
    mx_ref[:, :] = jnp.max(y3g, axis=1)
    mn_ref[:, :] = jnp.min(y3g, axis=1)
    ps = jnp.sum(y3.reshape(64, 8, 256), axis=0)
    pq = jnp.sum((y3 * y3).reshape(64, 8, 256), axis=0)

    @pl.when(i == 0)
    def _():
        sum3_ref[:, :] = jnp.zeros_like(sum3_ref)
        sq3_ref[:, :] = jnp.zeros_like(sq3_ref)

    sum3_ref[:, :] += ps
    sq3_ref[:, :] += pq


def _mlp3(x1, s2, q2, g2, beta2, w2t, w3t):
    return pl.pallas_call(
        _mlp3_body,
        grid=(GRID,),
        in_specs=[
            pl.BlockSpec((TILE, 128), lambda i: (i, 0)),
            pl.BlockSpec((8, 256), lambda i: (0, 0)),
            pl.BlockSpec((8, 256), lambda i: (0, 0)),
            pl.BlockSpec((1, 256), lambda i: (0, 0)),
            pl.BlockSpec((1, 256), lambda i: (0, 0)),
            pl.BlockSpec((128, 256), lambda i: (0, 0)),
            pl.BlockSpec((256, 256), lambda i: (0, 0)),
        ],
        out_specs=[
            pl.BlockSpec((8, 256), lambda i: (i, 0)),
            pl.BlockSpec((8, 256), lambda i: (i, 0)),
            pl.BlockSpec((8, 256), lambda i: (0, 0)),
            pl.BlockSpec((8, 256), lambda i: (0, 0)),
        ],
        out_shape=[
            jax.ShapeDtypeStruct((B * NPOINT, 256), jnp.float32),
            jax.ShapeDtypeStruct((B * NPOINT, 256), jnp.float32),
            jax.ShapeDtypeStruct((8, 256), jnp.float32),
            jax.ShapeDtypeStruct((8, 256), jnp.float32),
        ],
    )(x1, s2, q2, g2.reshape(1, 256), beta2.reshape(1, 256), w2t, w3t)


def _fin_body(mx_ref, mn_ref, sum3_ref, sq3_ref, ga_ref, be_ref, o_ref):
    scale, shift = _affine(sum3_ref[:, :], sq3_ref[:, :], ga_ref[:, :], be_ref[:, :])
    hi = jnp.maximum(mx_ref[:, :] * scale + shift, 0.0)
    lo = jnp.maximum(mn_ref[:, :] * scale + shift, 0.0)
    o_ref[:, :] = jnp.where(scale > 0.0, hi, lo)


def _finalize(mx, mn, s3, q3, g3, beta3):
    return pl.pallas_call(
        _fin_body,
        grid=(B * NPOINT // TILE,),
        in_specs=[
            pl.BlockSpec((TILE, 256), lambda i: (i, 0)),
            pl.BlockSpec((TILE, 256), lambda i: (i, 0)),
            pl.BlockSpec((8, 256), lambda i: (0, 0)),
            pl.BlockSpec((8, 256), lambda i: (0, 0)),
            pl.BlockSpec((1, 256), lambda i: (0, 0)),
            pl.BlockSpec((1, 256), lambda i: (0, 0)),
        ],
        out_specs=pl.BlockSpec((TILE, 256), lambda i: (i, 0)),
        out_shape=jax.ShapeDtypeStruct((B * NPOINT, 256), jnp.float32),
    )(mx, mn, s3, q3, g3.reshape(1, 256), beta3.reshape(1, 256))


def _ball_query_idx(new_xyz, xyz):
    d2 = jnp.sum((new_xyz[:, :, None, :] - xyz[:, None, :, :]) ** 2, axis=-1)
    n = xyz.shape[1]
    scores = jnp.where(d2 < RADIUS * RADIUS,
                       jnp.arange(n, dtype=jnp.int32)[None, None, :], jnp.int32(n))
    neg_vals, _ = jax.lax.top_k(-scores, NSAMPLE)
    idx = -neg_vals
    idx = jnp.where(idx >= n, 0, idx)
    return idx


def kernel(xyz, features, W1, b1, g1, beta1, W2, b2, g2, beta2, W3, b3, g3, beta3):
    fidx = _fps_pallas(xyz)
    new_xyz = jax.vmap(lambda p, i: p[i])(xyz, fidx)
    idx = _ball_query_idx(new_xyz, xyz)

    p_in = jnp.concatenate(
        [jnp.transpose(features, (0, 2, 1)), xyz,
         jnp.zeros((B, N, 5), jnp.float32)], axis=2)
    w1pad = jnp.concatenate(
        [jnp.transpose(W1[:, 3:131]), jnp.transpose(W1[:, 0:3]),
         jnp.zeros((5, 128), jnp.float32)], axis=0)
    c1 = _ptconv(p_in, w1pad, b1)

    nx_pad = jnp.concatenate(
        [new_xyz.reshape(B * NPOINT, 3),
         jnp.zeros((B * NPOINT, 5), jnp.float32)], axis=1)
    w1xpad = jnp.concatenate(
        [jnp.transpose(W1[:, 0:3]), jnp.zeros((5, 128), jnp.float32)], axis=0)
    u = _u_mat(nx_pad, w1xpad)

    g = jnp.take_along_axis(
        c1, idx.reshape(B, NPOINT * NSAMPLE, 1), axis=1).reshape(M, 128)

    s1, q1 = _stats1(g, u)
    w2t = jnp.transpose(W2)
    w3t = jnp.transpose(W3)
    x1, s2, q2 = _mlp2(g, u, s1, q1, g1, beta1, w2t)
    mx, mn, s3, q3 = _mlp3(x1, s2, q2, g2, beta2, w2t, w3t)
    out = _finalize(mx, mn, s3, q3, g3, beta3)
    new_features = jnp.transpose(out.reshape(B, NPOINT, 256), (0, 2, 1))
    return new_xyz, new_features


# speedup vs baseline: 9.0777x; 4.5262x over previous
import functools

import jax
import jax.numpy as jnp
import numpy as np
from jax import lax
from jax.experimental import pallas as pl
from jax.experimental.pallas import tpu as pltpu
from jax.experimental.pallas import tpu_sc as plsc

B = 8
N = 4096
NPOINT = 1024
RADIUS = 0.4
NSAMPLE = 64
R2 = np.float32(RADIUS * RADIUS)
M = B * NPOINT * NSAMPLE
TILE = 512
GRID = M // TILE
EPS = 1e-5


def _fps_body(xs_ref, ys_ref, zs_ref, out_ref):
    nb, n = xs_ref.shape
    xs = xs_ref[:, :]
    ys = ys_ref[:, :]
    zs = zs_ref[:, :]
    iota = jax.lax.broadcasted_iota(jnp.int32, (nb, n), 1)

    def step(k, carry):
        dists, f = carry
        out_ref[pl.ds(k, 1), :] = jnp.transpose(f)
        sel = iota == f
        cx = jnp.sum(jnp.where(sel, xs, 0.0), axis=1, keepdims=True)
        cy = jnp.sum(jnp.where(sel, ys, 0.0), axis=1, keepdims=True)
        cz = jnp.sum(jnp.where(sel, zs, 0.0), axis=1, keepdims=True)
        dx = xs - cx
        dy = ys - cy
        dz = zs - cz
        d = dx * dx + dy * dy + dz * dz
        dists = jnp.minimum(dists, d)
        m = jnp.max(dists, axis=1, keepdims=True)
        fn = jnp.min(jnp.where(dists == m, iota, n), axis=1, keepdims=True)
        return dists, fn.astype(jnp.int32)

    dists0 = jnp.full((nb, n), 1e10, dtype=jnp.float32)
    f0 = jnp.zeros((nb, 1), dtype=jnp.int32)
    jax.lax.fori_loop(0, NPOINT, step, (dists0, f0))


def _fps_pallas(xyz):
    out = pl.pallas_call(
        _fps_body,
        out_shape=jax.ShapeDtypeStruct((NPOINT, B), jnp.int32),
    )(xyz[:, :, 0], xyz[:, :, 1], xyz[:, :, 2])
    return jnp.transpose(out)


def _ptconv_body(x_ref, w_ref, b_ref, o_ref):
    o_ref[0] = jnp.dot(x_ref[0], w_ref[:, :],
                       preferred_element_type=jnp.float32) + b_ref[:, :]


def _ptconv(p_in, w1pad, b1):
    return pl.pallas_call(
        _ptconv_body,
        grid=(B, N // TILE),
        in_specs=[
            pl.BlockSpec((1, TILE, 136), lambda b, i: (b, i, 0)),
            pl.BlockSpec((136, 128), lambda b, i: (0, 0)),
            pl.BlockSpec((1, 128), lambda b, i: (0, 0)),
        ],
        out_specs=pl.BlockSpec((1, TILE, 128), lambda b, i: (b, i, 0)),
        out_shape=jax.ShapeDtypeStruct((B, N, 128), jnp.float32),
    )(p_in, w1pad, b1.reshape(1, 128))


def _u_body(x_ref, w_ref, o_ref):
    o_ref[:, :] = jnp.dot(x_ref[:, :], w_ref[:, :],
                          preferred_element_type=jnp.float32)


def _u_mat(nx_pad, w1xpad):
    return pl.pallas_call(
        _u_body,
        grid=(B * NPOINT // TILE,),
        in_specs=[
            pl.BlockSpec((TILE, 8), lambda i: (i, 0)),
            pl.BlockSpec((8, 128), lambda i: (0, 0)),
        ],
        out_specs=pl.BlockSpec((TILE, 128), lambda i: (i, 0)),
        out_shape=jax.ShapeDtypeStruct((B * NPOINT, 128), jnp.float32),
    )(nx_pad, w1xpad)


def _stats1_body(g_ref, u_ref, sum_ref, sq_ref):
    i = pl.program_id(0)
    y1 = (g_ref[:, :].reshape(8, NSAMPLE, 128)
          - u_ref[:, :].reshape(8, 1, 128)).reshape(TILE, 128)
    ps = jnp.sum(y1.reshape(64, 8, 128), axis=0)
    pq = jnp.sum((y1 * y1).reshape(64, 8, 128), axis=0)

    @pl.when(i == 0)
    def _():
        sum_ref[:, :] = jnp.zeros_like(sum_ref)
        sq_ref[:, :] = jnp.zeros_like(sq_ref)

    sum_ref[:, :] += ps
    sq_ref[:, :] += pq


def _stats1(g, u):
    return pl.pallas_call(
        _stats1_body,
        grid=(GRID,),
        in_specs=[
            pl.BlockSpec((TILE, 128), lambda i: (i, 0)),
            pl.BlockSpec((8, 128), lambda i: (i, 0)),
        ],
        out_specs=[
            pl.BlockSpec((8, 128), lambda i: (0, 0)),
            pl.BlockSpec((8, 128), lambda i: (0, 0)),
        ],
        out_shape=[jax.ShapeDtypeStruct((8, 128), jnp.float32)] * 2,
    )(g, u)


def _affine(sum_, sq_, gamma, beta):
    mean = jnp.sum(sum_, axis=0, keepdims=True) / M
    var = jnp.sum(sq_, axis=0, keepdims=True) / M - mean * mean
    scale = gamma.reshape(1, -1) / jnp.sqrt(var + EPS)
    shift = beta.reshape(1, -1) - mean * scale
    return scale, shift


def _mlp2_body(g_ref, u_ref, sum_ref, sq_ref, ga_ref, be_ref, w2_ref,
               x1_ref, sum2_ref, sq2_ref):
    i = pl.program_id(0)
    scale, shift = _affine(sum_ref[:, :], sq_ref[:, :], ga_ref[:, :], be_ref[:, :])
    y1 = (g_ref[:, :].reshape(8, NSAMPLE, 128)
          - u_ref[:, :].reshape(8, 1, 128)).reshape(TILE, 128)
    x1 = jnp.maximum(y1 * scale + shift, 0.0)
    x1_ref[:, :] = x1
    y2 = jnp.dot(x1, w2_ref[:, :], preferred_element_type=jnp.float32)
    ps = jnp.sum(y2.reshape(64, 8, 256), axis=0)
    pq = jnp.sum((y2 * y2).reshape(64, 8, 256), axis=0)

    @pl.when(i == 0)
    def _():
        sum2_ref[:, :] = jnp.zeros_like(sum2_ref)
        sq2_ref[:, :] = jnp.zeros_like(sq2_ref)

    sum2_ref[:, :] += ps
    sq2_ref[:, :] += pq


def _mlp2(g, u, s1, q1, g1, beta1, w2t):
    return pl.pallas_call(
        _mlp2_body,
        grid=(GRID,),
        in_specs=[
            pl.BlockSpec((TILE, 128), lambda i: (i, 0)),
            pl.BlockSpec((8, 128), lambda i: (i, 0)),
            pl.BlockSpec((8, 128), lambda i: (0, 0)),
            pl.BlockSpec((8, 128), lambda i: (0, 0)),
            pl.BlockSpec((1, 128), lambda i: (0, 0)),
            pl.BlockSpec((1, 128), lambda i: (0, 0)),
            pl.BlockSpec((128, 256), lambda i: (0, 0)),
        ],
        out_specs=[
            pl.BlockSpec((TILE, 128), lambda i: (i, 0)),
            pl.BlockSpec((8, 256), lambda i: (0, 0)),
            pl.BlockSpec((8, 256), lambda i: (0, 0)),
        ],
        out_shape=[
            jax.ShapeDtypeStruct((M, 128), jnp.float32),
            jax.ShapeDtypeStruct((8, 256), jnp.float32),
            jax.ShapeDtypeStruct((8, 256), jnp.float32),
        ],
    )(g, u, s1, q1, g1.reshape(1, 128), beta1.reshape(1, 128), w2t)


def _mlp3_body(x1_ref, sum2_ref, sq2_ref, ga_ref, be_ref, w2_ref, w3_ref,
               mx_ref, mn_ref, sum3_ref, sq3_ref):
    i = pl.program_id(0)
    scale, shift = _affine(sum2_ref[:, :], sq2_ref[:, :], ga_ref[:, :], be_ref[:, :])
    y2 = jnp.dot(x1_ref[:, :], w2_ref[:, :], preferred_element_type=jnp.float32)
    x2 = jnp.maximum(y2 * scale + shift, 0.0)
    y3 = jnp.dot(x2, w3_ref[:, :], preferred_element_type=jnp.float32)
    y3g = y3.reshape(8, NSAMPLE, 256)
    mx_ref[:, :] = jnp.max(y3g, axis=1)
    mn_ref[:, :] = jnp.min(y3g, axis=1)
    ps = jnp.sum(y3.reshape(64, 8, 256), axis=0)
    pq = jnp.sum((y3 * y3).reshape(64, 8, 256), axis=0)

    @pl.when(i == 0)
    def _():
        sum3_ref[:, :] = jnp.zeros_like(sum3_ref)
        sq3_ref[:, :] = jnp.zeros_like(sq3_ref)

    sum3_ref[:, :] += ps
    sq3_ref[:, :] += pq


def _mlp3(x1, s2, q2, g2, beta2, w2t, w3t):
    return pl.pallas_call(
        _mlp3_body,
        grid=(GRID,),
        in_specs=[
            pl.BlockSpec((TILE, 128), lambda i: (i, 0)),
            pl.BlockSpec((8, 256), lambda i: (0, 0)),
            pl.BlockSpec((8, 256), lambda i: (0, 0)),
            pl.BlockSpec((1, 256), lambda i: (0, 0)),
            pl.BlockSpec((1, 256), lambda i: (0, 0)),
            pl.BlockSpec((128, 256), lambda i: (0, 0)),
            pl.BlockSpec((256, 256), lambda i: (0, 0)),
        ],
        out_specs=[
            pl.BlockSpec((8, 256), lambda i: (i, 0)),
            pl.BlockSpec((8, 256), lambda i: (i, 0)),
            pl.BlockSpec((8, 256), lambda i: (0, 0)),
            pl.BlockSpec((8, 256), lambda i: (0, 0)),
        ],
        out_shape=[
            jax.ShapeDtypeStruct((B * NPOINT, 256), jnp.float32),
            jax.ShapeDtypeStruct((B * NPOINT, 256), jnp.float32),
            jax.ShapeDtypeStruct((8, 256), jnp.float32),
            jax.ShapeDtypeStruct((8, 256), jnp.float32),
        ],
    )(x1, s2, q2, g2.reshape(1, 256), beta2.reshape(1, 256), w2t, w3t)


def _fin_body(mx_ref, mn_ref, sum3_ref, sq3_ref, ga_ref, be_ref, o_ref):
    scale, shift = _affine(sum3_ref[:, :], sq3_ref[:, :], ga_ref[:, :], be_ref[:, :])
    hi = jnp.maximum(mx_ref[:, :] * scale + shift, 0.0)
    lo = jnp.maximum(mn_ref[:, :] * scale + shift, 0.0)
    o_ref[:, :] = jnp.where(scale > 0.0, hi, lo)


def _finalize(mx, mn, s3, q3, g3, beta3):
    return pl.pallas_call(
        _fin_body,
        grid=(B * NPOINT // TILE,),
        in_specs=[
            pl.BlockSpec((TILE, 256), lambda i: (i, 0)),
            pl.BlockSpec((TILE, 256), lambda i: (i, 0)),
            pl.BlockSpec((8, 256), lambda i: (0, 0)),
            pl.BlockSpec((8, 256), lambda i: (0, 0)),
            pl.BlockSpec((1, 256), lambda i: (0, 0)),
            pl.BlockSpec((1, 256), lambda i: (0, 0)),
        ],
        out_specs=pl.BlockSpec((TILE, 256), lambda i: (i, 0)),
        out_shape=jax.ShapeDtypeStruct((B * NPOINT, 256), jnp.float32),
    )(mx, mn, s3, q3, g3.reshape(1, 256), beta3.reshape(1, 256))


_ROWS = NPOINT // 4
_NV = N // 16


def _bq_row(p, xs_v, ys_v, zs_v, nx_v, ny_v, nz_v, idx_v, base):
    cx = jnp.full((16,), nx_v[pl.ds(p, 16)][0], jnp.float32)
    cy = jnp.full((16,), ny_v[pl.ds(p, 16)][0], jnp.float32)
    cz = jnp.full((16,), nz_v[pl.ds(p, 16)][0], jnp.float32)
    pad = jnp.full((16,), base, jnp.int32)
    idx_v[pl.ds(0, 16)] = pad
    idx_v[pl.ds(16, 16)] = pad
    idx_v[pl.ds(32, 16)] = pad
    idx_v[pl.ds(48, 16)] = pad
    lanes = lax.iota(jnp.int32, 16)

    def inner(i, cnt):
        off = pl.multiple_of(i * 16, 16)
        xv = xs_v[pl.ds(off, 16)]
        yv = ys_v[pl.ds(off, 16)]
        zv = zs_v[pl.ds(off, 16)]
        dx = xv - cx
        dy = yv - cy
        dz = zv - cz
        d2 = dx * dx + dy * dy + dz * dz
        m = d2 < R2
        vidx = lanes + (off + base)
        key = jnp.where(m, lanes, lanes + 16)
        _, sidx = plsc.sort_key_val(key, vidx)
        idx_v[pl.ds(cnt, 16)] = sidx
        return cnt + plsc.all_reduce_population_count(m)[0]

    cnt = lax.fori_loop(0, _NV, inner, jnp.int32(0))
    idx_v[pl.ds(cnt, 16)] = pad


def _bq_gather_body(xs_h, ys_h, zs_h, nx_h, ny_h, nz_h, c1_h, g_h,
                    xs_v, ys_v, zs_v, nx_v, ny_v, nz_v,
                    idx_a, idx_b, gb_a, gb_b, sg_a, sg_b, so_a, so_b):
    cid = lax.axis_index("c")
    sid = lax.axis_index("s")
    wid = sid * 2 + cid
    b = wid // 4
    pb = (wid % 4) * _ROWS
    gp0 = wid * _ROWS
    pltpu.sync_copy(xs_h.at[b], xs_v)
    pltpu.sync_copy(ys_h.at[b], ys_v)
    pltpu.sync_copy(zs_h.at[b], zs_v)
    pltpu.sync_copy(nx_h.at[b, pl.ds(pb, _ROWS)], nx_v.at[pl.ds(0, _ROWS)])
    pltpu.sync_copy(ny_h.at[b, pl.ds(pb, _ROWS)], ny_v.at[pl.ds(0, _ROWS)])
    pltpu.sync_copy(nz_h.at[b, pl.ds(pb, _ROWS)], nz_v.at[pl.ds(0, _ROWS)])
    base = b * N

    def row(p, idx_v):
        _bq_row(p, xs_v, ys_v, zs_v, nx_v, ny_v, nz_v, idx_v, base)

    def gather_start(idx_v, gb, sg):
        pltpu.async_copy(c1_h.at[idx_v.at[pl.ds(0, NSAMPLE)]], gb, sg)

    def gather_wait(gb, sg):
        pltpu.make_async_copy(c1_h.at[idx_a.at[pl.ds(0, NSAMPLE)]], gb, sg).wait()

    def out_start(p, gb, so):
        pltpu.async_copy(gb, g_h.at[gp0 + p], so)

    def out_wait(gb, so):
        pltpu.make_async_copy(gb, g_h.at[0], so).wait()

    row(0, idx_a)
    gather_start(idx_a, gb_a, sg_a)
    row(1, idx_b)
    gather_wait(gb_a, sg_a)
    out_start(0, gb_a, so_a)
    gather_start(idx_b, gb_b, sg_b)
    row(2, idx_a)
    gather_wait(gb_b, sg_b)
    out_start(1, gb_b, so_b)

    def pair(q, _):
        out_wait(gb_a, so_a)
        gather_start(idx_a, gb_a, sg_a)
        row(2 * q + 1, idx_b)
        gather_wait(gb_a, sg_a)
        out_start(2 * q, gb_a, so_a)
        out_wait(gb_b, so_b)
        gather_start(idx_b, gb_b, sg_b)
        row(2 * q + 2, idx_a)
        gather_wait(gb_b, sg_b)
        out_start(2 * q + 1, gb_b, so_b)
        return 0

    lax.fori_loop(1, _ROWS // 2 - 1, pair, 0)
    out_wait(gb_a, so_a)
    gather_start(idx_a, gb_a, sg_a)
    row(_ROWS - 1, idx_b)
    gather_wait(gb_a, sg_a)
    out_start(_ROWS - 2, gb_a, so_a)
    out_wait(gb_b, so_b)
    gather_start(idx_b, gb_b, sg_b)
    gather_wait(gb_b, sg_b)
    out_start(_ROWS - 1, gb_b, so_b)
    out_wait(gb_a, so_a)
    out_wait(gb_b, so_b)


def _bq_gather(xyz, new_xyz, c1):
    mesh = plsc.VectorSubcoreMesh(core_axis_name="c", subcore_axis_name="s")
    f = pl.kernel(
        _bq_gather_body,
        out_type=jax.ShapeDtypeStruct((B * NPOINT, NSAMPLE, 128), jnp.float32),
        mesh=mesh,
        compiler_params=pltpu.CompilerParams(needs_layout_passes=False),
        scratch_types=[
            pltpu.VMEM((N,), jnp.float32),
            pltpu.VMEM((N,), jnp.float32),
            pltpu.VMEM((N,), jnp.float32),
            pltpu.VMEM((_ROWS + 16,), jnp.float32),
            pltpu.VMEM((_ROWS + 16,), jnp.float32),
            pltpu.VMEM((_ROWS + 16,), jnp.float32),
            pltpu.VMEM((N + 16,), jnp.int32),
            pltpu.VMEM((N + 16,), jnp.int32),
            pltpu.VMEM((NSAMPLE, 128), jnp.float32),
            pltpu.VMEM((NSAMPLE, 128), jnp.float32),
            pltpu.SemaphoreType.DMA,
            pltpu.SemaphoreType.DMA,
            pltpu.SemaphoreType.DMA,
            pltpu.SemaphoreType.DMA,
        ],
    )
    return f(xyz[:, :, 0], xyz[:, :, 1], xyz[:, :, 2],
             new_xyz[:, :, 0], new_xyz[:, :, 1], new_xyz[:, :, 2],
             c1.reshape(B * N, 128))


def kernel(xyz, features, W1, b1, g1, beta1, W2, b2, g2, beta2, W3, b3, g3, beta3):
    fidx = _fps_pallas(xyz)
    new_xyz = jax.vmap(lambda p, i: p[i])(xyz, fidx)

    p_in = jnp.concatenate(
        [jnp.transpose(features, (0, 2, 1)), xyz,
         jnp.zeros((B, N, 5), jnp.float32)], axis=2)
    w1pad = jnp.concatenate(
        [jnp.transpose(W1[:, 3:131]), jnp.transpose(W1[:, 0:3]),
         jnp.zeros((5, 128), jnp.float32)], axis=0)
    c1 = _ptconv(p_in, w1pad, b1)

    nx_pad = jnp.concatenate(
        [new_xyz.reshape(B * NPOINT, 3),
         jnp.zeros((B * NPOINT, 5), jnp.float32)], axis=1)
    w1xpad = jnp.concatenate(
        [jnp.transpose(W1[:, 0:3]), jnp.zeros((5, 128), jnp.float32)], axis=0)
    u = _u_mat(nx_pad, w1xpad)

    g = _bq_gather(xyz, new_xyz, c1).reshape(M, 128)

    s1, q1 = _stats1(g, u)
    w2t = jnp.transpose(W2)
    w3t = jnp.transpose(W3)
    x1, s2, q2 = _mlp2(g, u, s1, q1, g1, beta1, w2t)
    mx, mn, s3, q3 = _mlp3(x1, s2, q2, g2, beta2, w2t, w3t)
    out = _finalize(mx, mn, s3, q3, g3, beta3)
    new_features = jnp.transpose(out.reshape(B, NPOINT, 256), (0, 2, 1))
    return new_xyz, new_features


# speedup vs baseline: 9.0795x; 1.0002x over previous
import functools

import jax
import jax.numpy as jnp
import numpy as np
from jax import lax
from jax.experimental import pallas as pl
from jax.experimental.pallas import tpu as pltpu
from jax.experimental.pallas import tpu_sc as plsc

B = 8
N = 4096
NPOINT = 1024
RADIUS = 0.4
NSAMPLE = 64
R2 = np.float32(RADIUS * RADIUS)
M = B * NPOINT * NSAMPLE
TILE = 512
GRID = M // TILE
EPS = 1e-5


def _fps_body(xs_ref, ys_ref, zs_ref, out_ref):
    nb, n = xs_ref.shape
    xs = xs_ref[:, :]
    ys = ys_ref[:, :]
    zs = zs_ref[:, :]
    iota = jax.lax.broadcasted_iota(jnp.int32, (nb, n), 1)

    def step(k, carry):
        dists, f = carry
        out_ref[pl.ds(k, 1), :] = jnp.transpose(f)
        sel = iota == f
        cx = jnp.sum(jnp.where(sel, xs, 0.0), axis=1, keepdims=True)
        cy = jnp.sum(jnp.where(sel, ys, 0.0), axis=1, keepdims=True)
        cz = jnp.sum(jnp.where(sel, zs, 0.0), axis=1, keepdims=True)
        dx = xs - cx
        dy = ys - cy
        dz = zs - cz
        d = dx * dx + dy * dy + dz * dz
        dists = jnp.minimum(dists, d)
        m = jnp.max(dists, axis=1, keepdims=True)
        fn = jnp.min(jnp.where(dists == m, iota, n), axis=1, keepdims=True)
        return dists, fn.astype(jnp.int32)

    dists0 = jnp.full((nb, n), 1e10, dtype=jnp.float32)
    f0 = jnp.zeros((nb, 1), dtype=jnp.int32)
    jax.lax.fori_loop(0, NPOINT, step, (dists0, f0))


def _fps_pallas(xyz):
    out = pl.pallas_call(
        _fps_body,
        out_shape=jax.ShapeDtypeStruct((NPOINT, B), jnp.int32),
    )(xyz[:, :, 0], xyz[:, :, 1], xyz[:, :, 2])
    return jnp.transpose(out)


def _ptconv_body(x_ref, w_ref, b_ref, o_ref):
    o_ref[0] = jnp.dot(x_ref[0], w_ref[:, :],
                       preferred_element_type=jnp.float32) + b_ref[:, :]


def _ptconv(p_in, w1pad, b1):
    return pl.pallas_call(
        _ptconv_body,
        grid=(B, N // TILE),
        in_specs=[
            pl.BlockSpec((1, TILE, 136), lambda b, i: (b, i, 0)),
            pl.BlockSpec((136, 128), lambda b, i: (0, 0)),
            pl.BlockSpec((1, 128), lambda b, i: (0, 0)),
        ],
        out_specs=pl.BlockSpec((1, TILE, 128), lambda b, i: (b, i, 0)),
        out_shape=jax.ShapeDtypeStruct((B, N, 128), jnp.float32),
    )(p_in, w1pad, b1.reshape(1, 128))


def _u_body(x_ref, w_ref, o_ref):
    o_ref[:, :] = jnp.dot(x_ref[:, :], w_ref[:, :],
                          preferred_element_type=jnp.float32)


def _u_mat(nx_pad, w1xpad):
    return pl.pallas_call(
        _u_body,
        grid=(B * NPOINT // TILE,),
        in_specs=[
            pl.BlockSpec((TILE, 8), lambda i: (i, 0)),
            pl.BlockSpec((8, 128), lambda i: (0, 0)),
        ],
        out_specs=pl.BlockSpec((TILE, 128), lambda i: (i, 0)),
        out_shape=jax.ShapeDtypeStruct((B * NPOINT, 128), jnp.float32),
    )(nx_pad, w1xpad)


def _stats1_body(g_ref, u_ref, sum_ref, sq_ref):
    i = pl.program_id(0)
    y1 = (g_ref[:, :].reshape(8, NSAMPLE, 128)
          - u_ref[:, :].reshape(8, 1, 128)).reshape(TILE, 128)
    ps = jnp.sum(y1.reshape(64, 8, 128), axis=0)
    pq = jnp.sum((y1 * y1).reshape(64, 8, 128), axis=0)

    @pl.when(i == 0)
    def _():
        sum_ref[:, :] = jnp.zeros_like(sum_ref)
        sq_ref[:, :] = jnp.zeros_like(sq_ref)

    sum_ref[:, :] += ps
    sq_ref[:, :] += pq


def _stats1(g, u):
    return pl.pallas_call(
        _stats1_body,
        grid=(GRID,),
        in_specs=[
            pl.BlockSpec((TILE, 128), lambda i: (i, 0)),
            pl.BlockSpec((8, 128), lambda i: (i, 0)),
        ],
        out_specs=[
            pl.BlockSpec((8, 128), lambda i: (0, 0)),
            pl.BlockSpec((8, 128), lambda i: (0, 0)),
        ],
        out_shape=[jax.ShapeDtypeStruct((8, 128), jnp.float32)] * 2,
    )(g, u)


def _affine(sum_, sq_, gamma, beta):
    mean = jnp.sum(sum_, axis=0, keepdims=True) / M
    var = jnp.sum(sq_, axis=0, keepdims=True) / M - mean * mean
    scale = gamma.reshape(1, -1) / jnp.sqrt(var + EPS)
    shift = beta.reshape(1, -1) - mean * scale
    return scale, shift


def _mlp2_body(g_ref, u_ref, sum_ref, sq_ref, ga_ref, be_ref, w2_ref,
               x1_ref, sum2_ref, sq2_ref):
    i = pl.program_id(0)
    scale, shift = _affine(sum_ref[:, :], sq_ref[:, :], ga_ref[:, :], be_ref[:, :])
    y1 = (g_ref[:, :].reshape(8, NSAMPLE, 128)
          - u_ref[:, :].reshape(8, 1, 128)).reshape(TILE, 128)
    x1 = jnp.maximum(y1 * scale + shift, 0.0)
    x1_ref[:, :] = x1
    y2 = jnp.dot(x1, w2_ref[:, :], preferred_element_type=jnp.float32)
    ps = jnp.sum(y2.reshape(64, 8, 256), axis=0)
    pq = jnp.sum((y2 * y2).reshape(64, 8, 256), axis=0)

    @pl.when(i == 0)
    def _():
        sum2_ref[:, :] = jnp.zeros_like(sum2_ref)
        sq2_ref[:, :] = jnp.zeros_like(sq2_ref)

    sum2_ref[:, :] += ps
    sq2_ref[:, :] += pq


def _mlp2(g, u, s1, q1, g1, beta1, w2t):
    return pl.pallas_call(
        _mlp2_body,
        grid=(GRID,),
        in_specs=[
            pl.BlockSpec((TILE, 128), lambda i: (i, 0)),
            pl.BlockSpec((8, 128), lambda i: (i, 0)),
            pl.BlockSpec((8, 128), lambda i: (0, 0)),
            pl.BlockSpec((8, 128), lambda i: (0, 0)),
            pl.BlockSpec((1, 128), lambda i: (0, 0)),
            pl.BlockSpec((1, 128), lambda i: (0, 0)),
            pl.BlockSpec((128, 256), lambda i: (0, 0)),
        ],
        out_specs=[
            pl.BlockSpec((TILE, 128), lambda i: (i, 0)),
            pl.BlockSpec((8, 256), lambda i: (0, 0)),
            pl.BlockSpec((8, 256), lambda i: (0, 0)),
        ],
        out_shape=[
            jax.ShapeDtypeStruct((M, 128), jnp.float32),
            jax.ShapeDtypeStruct((8, 256), jnp.float32),
            jax.ShapeDtypeStruct((8, 256), jnp.float32),
        ],
    )(g, u, s1, q1, g1.reshape(1, 128), beta1.reshape(1, 128), w2t)


def _mlp3_body(x1_ref, sum2_ref, sq2_ref, ga_ref, be_ref, w2_ref, w3_ref,
               mx_ref, mn_ref, sum3_ref, sq3_ref):
    i = pl.program_id(0)
    scale, shift = _affine(sum2_ref[:, :], sq2_ref[:, :], ga_ref[:, :], be_ref[:, :])
    y2 = jnp.dot(x1_ref[:, :], w2_ref[:, :], preferred_element_type=jnp.float32)
    x2 = jnp.maximum(y2 * scale + shift, 0.0)
    y3 = jnp.dot(x2, w3_ref[:, :], preferred_element_type=jnp.float32)
    y3g = y3.reshape(8, NSAMPLE, 256)
    mx_ref[:, :] = jnp.max(y3g, axis=1)
    mn_ref[:, :] = jnp.min(y3g, axis=1)
    ps = jnp.sum(y3.reshape(64, 8, 256), axis=0)
    pq = jnp.sum((y3 * y3).reshape(64, 8, 256), axis=0)

    @pl.when(i == 0)
    def _():
        sum3_ref[:, :] = jnp.zeros_like(sum3_ref)
        sq3_ref[:, :] = jnp.zeros_like(sq3_ref)

    sum3_ref[:, :] += ps
    sq3_ref[:, :] += pq


def _mlp3(x1, s2, q2, g2, beta2, w2t, w3t):
    return pl.pallas_call(
        _mlp3_body,
        grid=(GRID,),
        in_specs=[
            pl.BlockSpec((TILE, 128), lambda i: (i, 0)),
            pl.BlockSpec((8, 256), lambda i: (0, 0)),
            pl.BlockSpec((8, 256), lambda i: (0, 0)),
            pl.BlockSpec((1, 256), lambda i: (0, 0)),
            pl.BlockSpec((1, 256), lambda i: (0, 0)),
            pl.BlockSpec((128, 256), lambda i: (0, 0)),
            pl.BlockSpec((256, 256), lambda i: (0, 0)),
        ],
        out_specs=[
            pl.BlockSpec((8, 256), lambda i: (i, 0)),
            pl.BlockSpec((8, 256), lambda i: (i, 0)),
            pl.BlockSpec((8, 256), lambda i: (0, 0)),
            pl.BlockSpec((8, 256), lambda i: (0, 0)),
        ],
        out_shape=[
            jax.ShapeDtypeStruct((B * NPOINT, 256), jnp.float32),
            jax.ShapeDtypeStruct((B * NPOINT, 256), jnp.float32),
            jax.ShapeDtypeStruct((8, 256), jnp.float32),
            jax.ShapeDtypeStruct((8, 256), jnp.float32),
        ],
    )(x1, s2, q2, g2.reshape(1, 256), beta2.reshape(1, 256), w2t, w3t)


def _fin_body(mx_ref, mn_ref, sum3_ref, sq3_ref, ga_ref, be_ref, o_ref):
    scale, shift = _affine(sum3_ref[:, :], sq3_ref[:, :], ga_ref[:, :], be_ref[:, :])
    hi = jnp.maximum(mx_ref[:, :] * scale + shift, 0.0)
    lo = jnp.maximum(mn_ref[:, :] * scale + shift, 0.0)
    o_ref[:, :] = jnp.where(scale > 0.0, hi, lo)


def _finalize(mx, mn, s3, q3, g3, beta3):
    return pl.pallas_call(
        _fin_body,
        grid=(B * NPOINT // TILE,),
        in_specs=[
            pl.BlockSpec((TILE, 256), lambda i: (i, 0)),
            pl.BlockSpec((TILE, 256), lambda i: (i, 0)),
            pl.BlockSpec((8, 256), lambda i: (0, 0)),
            pl.BlockSpec((8, 256), lambda i: (0, 0)),
            pl.BlockSpec((1, 256), lambda i: (0, 0)),
            pl.BlockSpec((1, 256), lambda i: (0, 0)),
        ],
        out_specs=pl.BlockSpec((TILE, 256), lambda i: (i, 0)),
        out_shape=jax.ShapeDtypeStruct((B * NPOINT, 256), jnp.float32),
    )(mx, mn, s3, q3, g3.reshape(1, 256), beta3.reshape(1, 256))


_ROWS = NPOINT // 4
_NV = N // 16


def _bq_row(p, xs_v, ys_v, zs_v, nx_v, ny_v, nz_v, idx_v, base):
    cx = jnp.full((16,), nx_v[pl.ds(p, 16)][0], jnp.float32)
    cy = jnp.full((16,), ny_v[pl.ds(p, 16)][0], jnp.float32)
    cz = jnp.full((16,), nz_v[pl.ds(p, 16)][0], jnp.float32)
    pad = jnp.full((16,), base, jnp.int32)
    idx_v[pl.ds(0, 16)] = pad
    idx_v[pl.ds(16, 16)] = pad
    idx_v[pl.ds(32, 16)] = pad
    idx_v[pl.ds(48, 16)] = pad
    lanes = lax.iota(jnp.int32, 16)

    def inner(i, cnt):
        off = pl.multiple_of(i * 16, 16)
        xv = xs_v[pl.ds(off, 16)]
        yv = ys_v[pl.ds(off, 16)]
        zv = zs_v[pl.ds(off, 16)]
        dx = xv - cx
        dy = yv - cy
        dz = zv - cz
        d2 = dx * dx + dy * dy + dz * dz
        m = d2 < R2
        vidx = lanes + (off + base)
        rank = plsc.cumsum(jnp.where(m, jnp.int32(1), jnp.int32(0)))
        plsc.store_scatter(idx_v, [cnt + rank - 1], vidx, mask=m)
        return cnt + plsc.all_reduce_population_count(m)

    cnt_vec = lax.fori_loop(0, _NV, inner, jnp.zeros((16,), jnp.int32))
    idx_v[pl.ds(cnt_vec[0], 16)] = pad


def _bq_gather_body(xs_h, ys_h, zs_h, nx_h, ny_h, nz_h, c1_h, g_h,
                    xs_v, ys_v, zs_v, nx_v, ny_v, nz_v,
                    idx_a, idx_b, gb_a, gb_b, sg_a, sg_b, so_a, so_b):
    cid = lax.axis_index("c")
    sid = lax.axis_index("s")
    wid = sid * 2 + cid
    b = wid // 4
    pb = (wid % 4) * _ROWS
    gp0 = wid * _ROWS
    pltpu.sync_copy(xs_h.at[b], xs_v)
    pltpu.sync_copy(ys_h.at[b], ys_v)
    pltpu.sync_copy(zs_h.at[b], zs_v)
    pltpu.sync_copy(nx_h.at[b, pl.ds(pb, _ROWS)], nx_v.at[pl.ds(0, _ROWS)])
    pltpu.sync_copy(ny_h.at[b, pl.ds(pb, _ROWS)], ny_v.at[pl.ds(0, _ROWS)])
    pltpu.sync_copy(nz_h.at[b, pl.ds(pb, _ROWS)], nz_v.at[pl.ds(0, _ROWS)])
    base = b * N

    def row(p, idx_v):
        _bq_row(p, xs_v, ys_v, zs_v, nx_v, ny_v, nz_v, idx_v, base)

    def gather_start(idx_v, gb, sg):
        pltpu.async_copy(c1_h.at[idx_v.at[pl.ds(0, NSAMPLE)]], gb, sg)

    def gather_wait(gb, sg):
        pltpu.make_async_copy(c1_h.at[idx_a.at[pl.ds(0, NSAMPLE)]], gb, sg).wait()

    def out_start(p, gb, so):
        pltpu.async_copy(gb, g_h.at[gp0 + p], so)

    def out_wait(gb, so):
        pltpu.make_async_copy(gb, g_h.at[0], so).wait()

    row(0, idx_a)
    gather_start(idx_a, gb_a, sg_a)
    row(1, idx_b)
    gather_wait(gb_a, sg_a)
    out_start(0, gb_a, so_a)
    gather_start(idx_b, gb_b, sg_b)
    row(2, idx_a)
    gather_wait(gb_b, sg_b)
    out_start(1, gb_b, so_b)

    def pair(q, _):
        out_wait(gb_a, so_a)
        gather_start(idx_a, gb_a, sg_a)
        row(2 * q + 1, idx_b)
        gather_wait(gb_a, sg_a)
        out_start(2 * q, gb_a, so_a)
        out_wait(gb_b, so_b)
        gather_start(idx_b, gb_b, sg_b)
        row(2 * q + 2, idx_a)
        gather_wait(gb_b, sg_b)
        out_start(2 * q + 1, gb_b, so_b)
        return 0

    lax.fori_loop(1, _ROWS // 2 - 1, pair, 0)
    out_wait(gb_a, so_a)
    gather_start(idx_a, gb_a, sg_a)
    row(_ROWS - 1, idx_b)
    gather_wait(gb_a, sg_a)
    out_start(_ROWS - 2, gb_a, so_a)
    out_wait(gb_b, so_b)
    gather_start(idx_b, gb_b, sg_b)
    gather_wait(gb_b, sg_b)
    out_start(_ROWS - 1, gb_b, so_b)
    out_wait(gb_a, so_a)
    out_wait(gb_b, so_b)


def _bq_gather(xyz, new_xyz, c1):
    mesh = plsc.VectorSubcoreMesh(core_axis_name="c", subcore_axis_name="s")
    f = pl.kernel(
        _bq_gather_body,
        out_type=jax.ShapeDtypeStruct((B * NPOINT, NSAMPLE, 128), jnp.float32),
        mesh=mesh,
        compiler_params=pltpu.CompilerParams(needs_layout_passes=False),
        scratch_types=[
            pltpu.VMEM((N,), jnp.float32),
            pltpu.VMEM((N,), jnp.float32),
            pltpu.VMEM((N,), jnp.float32),
            pltpu.VMEM((_ROWS + 16,), jnp.float32),
            pltpu.VMEM((_ROWS + 16,), jnp.float32),
            pltpu.VMEM((_ROWS + 16,), jnp.float32),
            pltpu.VMEM((N + 16,), jnp.int32),
            pltpu.VMEM((N + 16,), jnp.int32),
            pltpu.VMEM((NSAMPLE, 128), jnp.float32),
            pltpu.VMEM((NSAMPLE, 128), jnp.float32),
            pltpu.SemaphoreType.DMA,
            pltpu.SemaphoreType.DMA,
            pltpu.SemaphoreType.DMA,
            pltpu.SemaphoreType.DMA,
        ],
    )
    return f(xyz[:, :, 0], xyz[:, :, 1], xyz[:, :, 2],
             new_xyz[:, :, 0], new_xyz[:, :, 1], new_xyz[:, :, 2],
             c1.reshape(B * N, 128))


def kernel(xyz, features, W1, b1, g1, beta1, W2, b2, g2, beta2, W3, b3, g3, beta3):
    fidx = _fps_pallas(xyz)
    new_xyz = jax.vmap(lambda p, i: p[i])(xyz, fidx)

    p_in = jnp.concatenate(
        [jnp.transpose(features, (0, 2, 1)), xyz,
         jnp.zeros((B, N, 5), jnp.float32)], axis=2)
    w1pad = jnp.concatenate(
        [jnp.transpose(W1[:, 3:131]), jnp.transpose(W1[:, 0:3]),
         jnp.zeros((5, 128), jnp.float32)], axis=0)
    c1 = _ptconv(p_in, w1pad, b1)

    nx_pad = jnp.concatenate(
        [new_xyz.reshape(B * NPOINT, 3),
         jnp.zeros((B * NPOINT, 5), jnp.float32)], axis=1)
    w1xpad = jnp.concatenate(
        [jnp.transpose(W1[:, 0:3]), jnp.zeros((5, 128), jnp.float32)], axis=0)
    u = _u_mat(nx_pad, w1xpad)

    g = _bq_gather(xyz, new_xyz, c1).reshape(M, 128)

    s1, q1 = _stats1(g, u)
    w2t = jnp.transpose(W2)
    w3t = jnp.transpose(W3)
    x1, s2, q2 = _mlp2(g, u, s1, q1, g1, beta1, w2t)
    mx, mn, s3, q3 = _mlp3(x1, s2, q2, g2, beta2, w2t, w3t)
    out = _finalize(mx, mn, s3, q3, g3, beta3)
    new_features = jnp.transpose(out.reshape(B, NPOINT, 256), (0, 2, 1))
    return new_xyz, new_features
